# baseline (speedup 1.0000x reference)
import functools

import jax
import jax.numpy as jnp
from jax import lax
from jax.experimental import pallas as pl
from jax.experimental.pallas import tpu as pltpu
from jax.experimental.pallas import tpu_sc as plsc

N = 10000
C = 128
LANES = 16
NC = 2
NS = 16
NW = NC * NS
K = 40
WPAD = 48
NBUF = 4
NSLOT = 8
NPAD = 10240
ROWS_PER_TILE = NPAD // NS
OUT_CHUNK = 40


def _bcast_lane(vec, l):
    idx = jnp.full((LANES, 1), l, jnp.int32)
    dnums = lax.GatherDimensionNumbers(
        offset_dims=(), collapsed_slice_dims=(0,), start_index_map=(0,))
    return lax.gather(vec, idx, dnums, slice_sizes=(1,),
                      mode=lax.GatherScatterMode.PROMISE_IN_BOUNDS)


def _zero_rows(rows_v, nrows):
    zv = jnp.zeros((LANES,), jnp.float32)

    def body(r, carry):
        for j in range(C // LANES):
            rows_v[r, pl.ds(j * LANES, LANES)] = zv
        return carry

    lax.fori_loop(0, nrows, body, 0)


def _sc_body(nchunk, h_hbm, src_hbm, dst_hbm, w_hbm, out_hbm,
             b0, b1, b2, b3, srcs, dsts, ws, acc_sh,
             g0, g1, g2, g3, s0, s1, s2, s3,
             i0, i1, i2, i3, i4, i5, i6, i7):
    c = lax.axis_index("c")
    s = lax.axis_index("s")
    wid = c * NS + s
    bufs = (b0, b1, b2, b3)
    gsem = (g0, g1, g2, g3)
    ssem = (s0, s1, s2, s3)
    isem = (i0, i1, i2, i3, i4, i5, i6, i7)
    ept = nchunk * K
    base = wid * ept

    _zero_rows(b0, OUT_CHUNK)
    for t in range(ROWS_PER_TILE // OUT_CHUNK):
        pltpu.sync_copy(b0,
                        acc_sh.at[pl.ds(s * ROWS_PER_TILE + t * OUT_CHUNK,
                                        OUT_CHUNK)])
    plsc.subcore_barrier()

    def issue_idx(i, slot):
        off = base + i * K
        pltpu.async_copy(src_hbm.at[pl.ds(off, K)], srcs.at[slot], isem[slot])
        pltpu.async_copy(dst_hbm.at[pl.ds(off, K)], dsts.at[slot], isem[slot])
        pltpu.async_copy(w_hbm.at[pl.ds(off, K)], ws.at[slot, pl.ds(0, K)],
                         isem[slot])

    def wait_idx(slot):
        sem = isem[slot]
        pltpu.make_async_copy(src_hbm.at[pl.ds(0, K)], srcs.at[slot],
                              sem).wait()
        pltpu.make_async_copy(dst_hbm.at[pl.ds(0, K)], dsts.at[slot],
                              sem).wait()
        pltpu.make_async_copy(w_hbm.at[pl.ds(0, K)],
                              ws.at[slot, pl.ds(0, K)], sem).wait()

    def issue_g(b, slot):
        pltpu.async_copy(h_hbm.at[srcs.at[slot]], bufs[b], gsem[b])

    def wait_g(b):
        pltpu.make_async_copy(h_hbm.at[srcs.at[0]], bufs[b], gsem[b]).wait()

    def issue_s(b, slot):
        pltpu.async_copy(bufs[b], acc_sh.at[dsts.at[slot]], ssem[b], add=True)

    def wait_s(b):
        pltpu.make_async_copy(bufs[b], acc_sh.at[dsts.at[0]], ssem[b]).wait()

    def scale(b, slot):
        buf = bufs[b]

        def ebody(e, carry):
            g = e // LANES
            wg = ws[slot, pl.ds(g * LANES, LANES)]
            wv = _bcast_lane(wg, e - g * LANES)
            for j in range(C // LANES):
                sl = pl.ds(j * LANES, LANES)
                buf[e, sl] = buf[e, sl] * wv
            return carry

        lax.fori_loop(0, K, ebody, 0, unroll=2)

    def iter_body(i, pmod, do_idx, do_g, do_ws):
        slot = pmod
        if do_idx:
            issue_idx(i + 3, (pmod + 3) % NSLOT)
        if do_g:
            if do_ws:
                wait_s((pmod + 2) % NBUF)
            wait_idx((pmod + 2) % NSLOT)
            issue_g((pmod + 2) % NBUF, (pmod + 2) % NSLOT)
        wait_g(pmod % NBUF)
        scale(pmod % NBUF, slot)
        issue_s(pmod % NBUF, slot)

    issue_idx(0, 0)
    issue_idx(1, 1)
    issue_idx(2, 2)
    wait_idx(0)
    issue_g(0, 0)
    wait_idx(1)
    issue_g(1, 1)
    iter_body(0, 0, True, True, False)
    iter_body(1, 1, True, True, False)

    def steady(t, carry):
        for p in range(NSLOT):
            iter_body(NSLOT * t + 2 + p, (2 + p) % NSLOT, True, True, True)
        return carry

    lax.fori_loop(0, (nchunk - 8) // NSLOT, steady, 0)

    for q in range(6, 0, -1):
        i = nchunk - q
        iter_body(i, i % NSLOT, q > 3, q > 2, q > 2)
    for b in range(NBUF):
        wait_s(b)
    plsc.subcore_barrier()

    for t in range(ROWS_PER_TILE // OUT_CHUNK):
        r0 = s * ROWS_PER_TILE + t * OUT_CHUNK
        pltpu.sync_copy(acc_sh.at[pl.ds(r0, OUT_CHUNK)], b0)
        pltpu.sync_copy(b0, out_hbm.at[pl.ds(c * NPAD + r0, OUT_CHUNK)])


def _make_sc_call(nchunk):
    mesh = plsc.VectorSubcoreMesh(core_axis_name="c", subcore_axis_name="s")
    return pl.kernel(
        functools.partial(_sc_body, nchunk),
        out_type=jax.ShapeDtypeStruct((NC * NPAD, C), jnp.float32),
        mesh=mesh,
        scratch_types=(
            [pltpu.VMEM((K, C), jnp.float32) for _ in range(NBUF)]
            + [pltpu.VMEM((NSLOT, K), jnp.int32),
               pltpu.VMEM((NSLOT, K), jnp.int32),
               pltpu.VMEM((NSLOT, WPAD), jnp.float32),
               pltpu.VMEM_SHARED((NPAD, C), jnp.float32)]
            + [pltpu.SemaphoreType.DMA] * (NBUF + NBUF + NSLOT)
        ),
    )


def _h_body(k_ref, b_ref, h_ref):
    h_ref[...] = k_ref[...] + b_ref[...]


_SELU_SCALE = 1.0507009873554804934193349852946
_SELU_ALPHA = 1.6732632423543772848170429916717


def _post_body(p0_ref, p1_ref, sk_ref, o_ref):
    z = p0_ref[...] + p1_ref[...] + sk_ref[...]
    neg = _SELU_ALPHA * (jnp.exp(jnp.minimum(z, 0.0)) - 1.0)
    o_ref[...] = _SELU_SCALE * jnp.where(z > 0.0, z, neg)


_BLK = 1000


def kernel(x, edge_index, edge_weight, kernel, bias, skip_weight):
    del x
    src = edge_index[0].astype(jnp.int32)
    dst = edge_index[1].astype(jnp.int32)
    w = edge_weight.astype(jnp.float32)

    e = src.shape[0]
    ept = -(-e // (NW * K * NSLOT)) * (K * NSLOT)
    e_pad = NW * ept
    nchunk = ept // K
    if e_pad != e:
        pad = e_pad - e
        zi = jnp.zeros((pad,), jnp.int32)
        src = jnp.concatenate([src, zi])
        dst = jnp.concatenate([dst, zi])
        w = jnp.concatenate([w, jnp.zeros((pad,), jnp.float32)])
    h = pl.pallas_call(
        _h_body,
        grid=(N // _BLK,),
        in_specs=[
            pl.BlockSpec((_BLK, C), lambda i: (i, 0)),
            pl.BlockSpec((1, C), lambda i: (0, 0)),
        ],
        out_specs=pl.BlockSpec((_BLK, C), lambda i: (i, 0)),
        out_shape=jax.ShapeDtypeStruct((N, C), jnp.float32),
    )(kernel, bias.reshape(1, C))

    p = _make_sc_call(nchunk)(h, src, dst, w)

    out = pl.pallas_call(
        _post_body,
        grid=(N // _BLK,),
        in_specs=[
            pl.BlockSpec((_BLK, C), lambda i: (i, 0)),
            pl.BlockSpec((_BLK, C), lambda i: (i, 0)),
            pl.BlockSpec((1, C), lambda i: (0, 0)),
        ],
        out_specs=pl.BlockSpec((_BLK, C), lambda i: (i, 0)),
        out_shape=jax.ShapeDtypeStruct((N, C), jnp.float32),
    )(p[:N], p[NPAD:NPAD + N], skip_weight.reshape(1, C))
    return out


# speedup vs baseline: 1.5977x; 1.5977x over previous
import functools

import jax
import jax.numpy as jnp
from jax import lax
from jax.experimental import pallas as pl
from jax.experimental.pallas import tpu as pltpu
from jax.experimental.pallas import tpu_sc as plsc

N = 10000
C = 128
LANES = 16
NC = 2
NS = 16
NW = NC * NS
K = 40
WPAD = 48
NBUF = 4
NSLOT = 8
NPAD = 10240
ROWS_PER_TILE = NPAD // NS
OUT_CHUNK = 40


def _bcast_lane(vec, l):
    idx = jnp.full((LANES, 1), l, jnp.int32)
    dnums = lax.GatherDimensionNumbers(
        offset_dims=(), collapsed_slice_dims=(0,), start_index_map=(0,))
    return lax.gather(vec, idx, dnums, slice_sizes=(1,),
                      mode=lax.GatherScatterMode.PROMISE_IN_BOUNDS)


def _zero_rows(rows_v, nrows):
    zv = jnp.zeros((LANES,), jnp.float32)

    def body(r, carry):
        for j in range(C // LANES):
            rows_v[r, pl.ds(j * LANES, LANES)] = zv
        return carry

    lax.fori_loop(0, nrows, body, 0)


def _sc_body(n0, n1, h_hbm, src_hbm, dst_hbm, w_hbm, out_hbm,
             b0, b1, b2, b3, srcs, dsts, ws, acc_sh,
             g0, g1, g2, g3, s0, s1, s2, s3,
             i0, i1, i2, i3, i4, i5, i6, i7):
    c = lax.axis_index("c")
    s = lax.axis_index("s")
    bufs = (b0, b1, b2, b3)
    gsem = (g0, g1, g2, g3)
    ssem = (s0, s1, s2, s3)
    isem = (i0, i1, i2, i3, i4, i5, i6, i7)
    is0 = c == 0
    nchunk = jnp.where(is0, n0, n1)
    base = jnp.where(is0, s * (n0 * K), NS * (n0 * K) + s * (n1 * K))

    _zero_rows(b0, OUT_CHUNK)
    for t in range(ROWS_PER_TILE // OUT_CHUNK):
        pltpu.sync_copy(b0,
                        acc_sh.at[pl.ds(s * ROWS_PER_TILE + t * OUT_CHUNK,
                                        OUT_CHUNK)])
    plsc.subcore_barrier()

    def issue_idx(i, slot):
        off = base + i * K
        pltpu.async_copy(src_hbm.at[pl.ds(off, K)], srcs.at[slot], isem[slot])
        pltpu.async_copy(dst_hbm.at[pl.ds(off, K)], dsts.at[slot], isem[slot])
        pltpu.async_copy(w_hbm.at[pl.ds(off, K)], ws.at[slot, pl.ds(0, K)],
                         isem[slot])

    def wait_idx(slot):
        sem = isem[slot]
        pltpu.make_async_copy(src_hbm.at[pl.ds(0, K)], srcs.at[slot],
                              sem).wait()
        pltpu.make_async_copy(dst_hbm.at[pl.ds(0, K)], dsts.at[slot],
                              sem).wait()
        pltpu.make_async_copy(w_hbm.at[pl.ds(0, K)],
                              ws.at[slot, pl.ds(0, K)], sem).wait()

    def issue_g(b, slot):
        pltpu.async_copy(h_hbm.at[srcs.at[slot]], bufs[b], gsem[b])

    def wait_g(b):
        pltpu.make_async_copy(h_hbm.at[srcs.at[0]], bufs[b], gsem[b]).wait()

    def issue_s(b, slot):
        pltpu.async_copy(bufs[b], acc_sh.at[dsts.at[slot]], ssem[b], add=True)

    def wait_s(b):
        pltpu.make_async_copy(bufs[b], acc_sh.at[dsts.at[0]], ssem[b]).wait()

    def scale(b, slot):
        buf = bufs[b]

        def ebody(e, carry):
            g = e // LANES
            wg = ws[slot, pl.ds(g * LANES, LANES)]
            wv = _bcast_lane(wg, e - g * LANES)
            for j in range(C // LANES):
                sl = pl.ds(j * LANES, LANES)
                buf[e, sl] = buf[e, sl] * wv
            return carry

        lax.fori_loop(0, K, ebody, 0, unroll=2)

    def iter_body(i, pmod, do_idx, do_g, do_ws):
        slot = pmod
        if do_idx:
            issue_idx(i + 3, (pmod + 3) % NSLOT)
        if do_g:
            if do_ws:
                wait_s((pmod + 2) % NBUF)
            wait_idx((pmod + 2) % NSLOT)
            issue_g((pmod + 2) % NBUF, (pmod + 2) % NSLOT)
        wait_g(pmod % NBUF)
        scale(pmod % NBUF, slot)
        issue_s(pmod % NBUF, slot)

    issue_idx(0, 0)
    issue_idx(1, 1)
    issue_idx(2, 2)
    wait_idx(0)
    issue_g(0, 0)
    wait_idx(1)
    issue_g(1, 1)
    iter_body(0, 0, True, True, False)
    iter_body(1, 1, True, True, False)

    def steady(t, carry):
        for p in range(NSLOT):
            iter_body(NSLOT * t + 2 + p, (2 + p) % NSLOT, True, True, True)
        return carry

    lax.fori_loop(0, lax.div(nchunk - 8, NSLOT), steady, 0)

    for q in range(6, 0, -1):
        i = nchunk - q
        iter_body(i, (NSLOT - q) % NSLOT, q > 3, q > 2, q > 2)
    for b in range(NBUF):
        wait_s(b)
    plsc.subcore_barrier()

    for t in range(ROWS_PER_TILE // OUT_CHUNK):
        r0 = s * ROWS_PER_TILE + t * OUT_CHUNK
        pltpu.sync_copy(acc_sh.at[pl.ds(r0, OUT_CHUNK)], b0)
        pltpu.sync_copy(b0, out_hbm.at[pl.ds(c * NPAD + r0, OUT_CHUNK)])


def _make_sc_call(n0, n1):
    mesh = plsc.VectorSubcoreMesh(core_axis_name="c", subcore_axis_name="s")
    return pl.kernel(
        functools.partial(_sc_body, n0, n1),
        out_type=jax.ShapeDtypeStruct((NC * NPAD, C), jnp.float32),
        mesh=mesh,
        scratch_types=(
            [pltpu.VMEM((K, C), jnp.float32) for _ in range(NBUF)]
            + [pltpu.VMEM((NSLOT, K), jnp.int32),
               pltpu.VMEM((NSLOT, K), jnp.int32),
               pltpu.VMEM((NSLOT, WPAD), jnp.float32),
               pltpu.VMEM_SHARED((NPAD, C), jnp.float32)]
            + [pltpu.SemaphoreType.DMA] * (NBUF + NBUF + NSLOT)
        ),
    )


def _h_body(k_ref, b_ref, h_ref):
    h_ref[...] = k_ref[...] + b_ref[...]


_SELU_SCALE = 1.0507009873554804934193349852946
_SELU_ALPHA = 1.6732632423543772848170429916717


def _post_body(p0_ref, p1_ref, sk_ref, o_ref):
    z = p0_ref[...] + p1_ref[...] + sk_ref[...]
    neg = _SELU_ALPHA * (jnp.exp(jnp.minimum(z, 0.0)) - 1.0)
    o_ref[...] = _SELU_SCALE * jnp.where(z > 0.0, z, neg)


_BLK = 1000
_CORE0_FRAC = 0.25


def kernel(x, edge_index, edge_weight, kernel, bias, skip_weight):
    del x
    src = edge_index[0].astype(jnp.int32)
    dst = edge_index[1].astype(jnp.int32)
    w = edge_weight.astype(jnp.float32)

    e = src.shape[0]
    unit = K * NSLOT
    units = -(-e // (NS * unit))
    e_pad = NS * units * unit
    u0 = max(1, round(units * _CORE0_FRAC))
    n0 = u0 * NSLOT
    n1 = (units - u0) * NSLOT
    if e_pad != e:
        pad = e_pad - e
        zi = jnp.zeros((pad,), jnp.int32)
        src = jnp.concatenate([src, zi])
        dst = jnp.concatenate([dst, zi])
        w = jnp.concatenate([w, jnp.zeros((pad,), jnp.float32)])
    h = pl.pallas_call(
        _h_body,
        grid=(N // _BLK,),
        in_specs=[
            pl.BlockSpec((_BLK, C), lambda i: (i, 0)),
            pl.BlockSpec((1, C), lambda i: (0, 0)),
        ],
        out_specs=pl.BlockSpec((_BLK, C), lambda i: (i, 0)),
        out_shape=jax.ShapeDtypeStruct((N, C), jnp.float32),
    )(kernel, bias.reshape(1, C))

    p = _make_sc_call(n0, n1)(h, src, dst, w)

    out = pl.pallas_call(
        _post_body,
        grid=(N // _BLK,),
        in_specs=[
            pl.BlockSpec((_BLK, C), lambda i: (i, 0)),
            pl.BlockSpec((_BLK, C), lambda i: (i, 0)),
            pl.BlockSpec((1, C), lambda i: (0, 0)),
        ],
        out_specs=pl.BlockSpec((_BLK, C), lambda i: (i, 0)),
        out_shape=jax.ShapeDtypeStruct((N, C), jnp.float32),
    )(p[:N], p[NPAD:NPAD + N], skip_weight.reshape(1, C))
    return out
